# baseline (speedup 1.0000x reference)
import functools

import jax
import jax.numpy as jnp
from jax import lax
from jax.experimental import pallas as pl
from jax.experimental.pallas import tpu as pltpu
from jax.experimental.pallas import tpu_sc as plsc

N = 10000
E = 160000
NF = 128
ED = 16
D = 128
B = 64

NW = 32
TILES = 16
CHUNK = 128
CPT = 40
E_PAD = NW * CPT * CHUNK
N_PAD = 10240
ROWS_PT = N_PAD // TILES
DST_PAD = N

NEG = -3.0e38


def _bdot(a, b):
    return jnp.dot(a.astype(jnp.bfloat16), b.astype(jnp.bfloat16),
                   preferred_element_type=jnp.float32)

def _eap_body(ea_ref, we_ref, b_ref, o_ref):
    o_ref[...] = _bdot(ea_ref[...], we_ref[...]) + b_ref[...]


def _eap(ea_pad, W_e, b2d):
    blk = 4096
    return pl.pallas_call(
        _eap_body,
        grid=(E_PAD // blk,),
        in_specs=[
            pl.BlockSpec((blk, ED), lambda i: (i, 0)),
            pl.BlockSpec((ED, D), lambda i: (0, 0)),
            pl.BlockSpec((1, D), lambda i: (0, 0)),
        ],
        out_specs=pl.BlockSpec((blk, D), lambda i: (i, 0)),
        out_shape=jax.ShapeDtypeStruct((E_PAD, D), jnp.float32),
    )(ea_pad, W_e, b2d)


def _lin0_body(x_ref, w_ref, b_ref, wx_ref, out_ref, y_ref):
    h = jnp.maximum(_bdot(x_ref[...], w_ref[...]) + b_ref[...], 0.0)
    out_ref[...] = h
    y_ref[...] = _bdot(h, wx_ref[...])


def _lin0(x, lin0_W, b2d, W_x):
    blk = 2000
    return pl.pallas_call(
        _lin0_body,
        grid=(N // blk,),
        in_specs=[
            pl.BlockSpec((blk, NF), lambda i: (i, 0)),
            pl.BlockSpec((NF, D), lambda i: (0, 0)),
            pl.BlockSpec((1, D), lambda i: (0, 0)),
            pl.BlockSpec((D, D), lambda i: (0, 0)),
        ],
        out_specs=[
            pl.BlockSpec((blk, D), lambda i: (i, 0)),
            pl.BlockSpec((blk, D), lambda i: (i, 0)),
        ],
        out_shape=[
            jax.ShapeDtypeStruct((N, D), jnp.float32),
            jax.ShapeDtypeStruct((N, D), jnp.float32),
        ],
    )(x, lin0_W, b2d, W_x)


def _edge_body(y_hbm, eap_hbm, src_hbm, dst_hbm, agg_hbm, deg_hbm,
               idx_s, idx_d, rows, eapb, ones, dzero, agg_sh, deg_sh,
               sem_g, sem_e):
    c = lax.axis_index("c")
    s = lax.axis_index("s")
    w = s * 2 + c
    t = s

    zero16 = jnp.zeros((16,), jnp.float32)

    for i in range(8):
        ones[pl.ds(i * 16, 16)] = zero16 + 1.0

    def _zrow(r, _):
        for c8 in range(8):
            rows[r, pl.ds(c8 * 16, 16)] = zero16
        return 0
    lax.fori_loop(0, CHUNK, _zrow, 0)

    def _zd(i, _):
        dzero[pl.ds(i * 16, 16)] = zero16
        return 0
    lax.fori_loop(0, ROWS_PT // 16, _zd, 0)

    for k in range(ROWS_PT // CHUNK):
        pltpu.sync_copy(rows, agg_sh.at[pl.ds(t * ROWS_PT + k * CHUNK, CHUNK)])
    pltpu.sync_copy(dzero, deg_sh.at[pl.ds(t * ROWS_PT, ROWS_PT)])
    plsc.subcore_barrier()

    pltpu.sync_copy(src_hbm.at[pl.ds(w * CPT, CPT)], idx_s)
    pltpu.sync_copy(dst_hbm.at[pl.ds(w * CPT, CPT)], idx_d)

    def _chunk(j, _):
        g = pltpu.async_copy(y_hbm.at[idx_s.at[j]], rows, sem_g)
        e = pltpu.async_copy(eap_hbm.at[pl.ds(w * (CPT * CHUNK) + j * CHUNK, CHUNK)],
                             eapb, sem_e)
        g.wait()
        e.wait()

        def _row(r, _):
            for c8 in range(8):
                sl = pl.ds(c8 * 16, 16)
                rows[r, sl] = jnp.maximum(rows[r, sl] + eapb[r, sl], 0.0)
            return 0
        lax.fori_loop(0, CHUNK, _row, 0)

        pltpu.sync_copy(rows, agg_sh.at[idx_d.at[j]], add=True)
        pltpu.sync_copy(ones, deg_sh.at[idx_d.at[j]], add=True)
        return 0

    lax.fori_loop(0, CPT, _chunk, 0)
    plsc.subcore_barrier()

    pltpu.sync_copy(agg_sh.at[pl.ds(t * ROWS_PT, ROWS_PT)],
                    agg_hbm.at[c, pl.ds(t * ROWS_PT, ROWS_PT)])
    pltpu.sync_copy(deg_sh.at[pl.ds(t * ROWS_PT, ROWS_PT)],
                    deg_hbm.at[c, pl.ds(t * ROWS_PT, ROWS_PT)])


@functools.cache
def _edge_call():
    return functools.partial(
        pl.kernel,
        out_type=(
            jax.ShapeDtypeStruct((2, N_PAD, D), jnp.float32),
            jax.ShapeDtypeStruct((2, N_PAD), jnp.float32),
        ),
        mesh=plsc.VectorSubcoreMesh(core_axis_name="c", subcore_axis_name="s",
                                    num_cores=2, num_subcores=TILES),
        scratch_types=[
        pltpu.VMEM((CPT, CHUNK), jnp.int32),
        pltpu.VMEM((CPT, CHUNK), jnp.int32),
        pltpu.VMEM((CHUNK, D), jnp.float32),
        pltpu.VMEM((CHUNK, D), jnp.float32),
        pltpu.VMEM((CHUNK,), jnp.float32),
        pltpu.VMEM((ROWS_PT,), jnp.float32),
        pltpu.VMEM_SHARED((N_PAD, D), jnp.float32),
        pltpu.VMEM_SHARED((N_PAD,), jnp.float32),
        pltpu.SemaphoreType.DMA,
        pltpu.SemaphoreType.DMA,
        ],
    )(_edge_body)


def _gru_body(agg_ref, deg_ref, out_ref, wi_ref, wh_ref, bi_ref, bh_ref,
              wx_ref, new_ref, y_ref):
    d = deg_ref[...]
    inv = 1.0 / jnp.maximum(d[:, 0:1] + d[:, 1:2], 1.0)
    agg = (agg_ref[0] + agg_ref[1]) * inv
    out = out_ref[...]
    gi = _bdot(agg, wi_ref[...]) + bi_ref[...]
    gh = _bdot(out, wh_ref[...]) + bh_ref[...]
    r = jax.nn.sigmoid(gi[:, :D] + gh[:, :D])
    z = jax.nn.sigmoid(gi[:, D:2 * D] + gh[:, D:2 * D])
    n = jnp.tanh(gi[:, 2 * D:] + r * gh[:, 2 * D:])
    new = (1.0 - z) * n + z * out
    new_ref[...] = new
    y_ref[...] = _bdot(new, wx_ref[...])


def _gru(agg2, deg_col, out, gru_Wi, gru_Wh, bi2d, bh2d, W_x):
    blk = 2000
    return pl.pallas_call(
        _gru_body,
        grid=(N // blk,),
        in_specs=[
            pl.BlockSpec((2, blk, D), lambda i: (0, i, 0)),
            pl.BlockSpec((blk, 2), lambda i: (i, 0)),
            pl.BlockSpec((blk, D), lambda i: (i, 0)),
            pl.BlockSpec((D, 3 * D), lambda i: (0, 0)),
            pl.BlockSpec((D, 3 * D), lambda i: (0, 0)),
            pl.BlockSpec((1, 3 * D), lambda i: (0, 0)),
            pl.BlockSpec((1, 3 * D), lambda i: (0, 0)),
            pl.BlockSpec((D, D), lambda i: (0, 0)),
        ],
        out_specs=[
            pl.BlockSpec((blk, D), lambda i: (i, 0)),
            pl.BlockSpec((blk, D), lambda i: (i, 0)),
        ],
        out_shape=[
            jax.ShapeDtypeStruct((N, D), jnp.float32),
            jax.ShapeDtypeStruct((N, D), jnp.float32),
        ],
    )(agg2, deg_col, out, gru_Wi, gru_Wh, bi2d, bh2d, W_x)


def _s2s_body(out_ref, bcol_ref, brow_ref, lwi_ref, lwh_ref, lb_ref,
              w1_ref, b1_ref, w2_ref, b2_ref, w3_ref, b3_ref, v_ref):
    out = out_ref[...]
    bcol = bcol_ref[...]
    brow = brow_ref[...]
    onehot = (bcol == lax.broadcasted_iota(jnp.int32, (N, B), 1)).astype(jnp.float32)
    onehot_T = (brow == lax.broadcasted_iota(jnp.int32, (B, N), 0)).astype(jnp.float32)

    q_star = jnp.zeros((B, 2 * D), jnp.float32)
    h_l = jnp.zeros((B, D), jnp.float32)
    c_l = jnp.zeros((B, D), jnp.float32)
    dn_last = (((1,), (1,)), ((), ()))
    for _ in range(6):
        g = (_bdot(q_star, lwi_ref[...]) + _bdot(h_l, lwh_ref[...])
             + lb_ref[...])
        c_l = (jax.nn.sigmoid(g[:, D:2 * D]) * c_l
               + jax.nn.sigmoid(g[:, :D]) * jnp.tanh(g[:, 2 * D:3 * D]))
        h_l = jax.nn.sigmoid(g[:, 3 * D:]) * jnp.tanh(c_l)
        q = h_l
        e_mat = lax.dot_general(out, q, dn_last,
                                preferred_element_type=jnp.float32,
                                precision=lax.Precision.HIGHEST)
        e = jnp.sum(e_mat * onehot, axis=1, keepdims=True)
        seg_max = jnp.max(jnp.where(onehot > 0.0, e_mat, NEG), axis=0,
                          keepdims=True)
        e_max_b = lax.dot_general(onehot, seg_max, dn_last,
                                  preferred_element_type=jnp.float32,
                                  precision=lax.Precision.HIGHEST)
        a = jnp.exp(e - e_max_b)
        a_den = jnp.sum(onehot * a, axis=0, keepdims=True)
        a = a / lax.dot_general(onehot, a_den, dn_last,
                                preferred_element_type=jnp.float32,
                                precision=lax.Precision.HIGHEST)
        r_read = jnp.dot(onehot_T, a * out,
                         preferred_element_type=jnp.float32,
                         precision=lax.Precision.HIGHEST)
        q_star = jnp.concatenate([q, r_read], axis=1)

    v = jnp.maximum(_bdot(q_star, w1_ref[...]) + b1_ref[...], 0.0)
    v = jnp.maximum(_bdot(v, w2_ref[...]) + b2_ref[...], 0.0)
    v_ref[...] = _bdot(v, w3_ref[...]) + b3_ref[...]


def _s2s(out, bcol, brow, lstm_Wi, lstm_Wh, lb2d, w1, b1, w2, b2, w3, b3):
    return pl.pallas_call(
        _s2s_body,
        out_shape=jax.ShapeDtypeStruct((B, 1), jnp.float32),
    )(out, bcol, brow, lstm_Wi, lstm_Wh, lb2d, w1, b1, w2, b2, w3, b3)


def kernel(x, edge_index, edge_attr, batch, lin0_W, lin0_b, msg_W, msg_b,
           gru_Wi, gru_Wh, gru_bi, gru_bh, lstm_Wi, lstm_Wh, lstm_b,
           mlp_W1, mlp_b1, mlp_W2, mlp_b2, mlp_W3, mlp_b3):
    src = edge_index[0].astype(jnp.int32)
    dst = edge_index[1].astype(jnp.int32)
    pad = E_PAD - E
    src2d = jnp.concatenate([src, jnp.zeros((pad,), jnp.int32)]).reshape(E_PAD // CHUNK, CHUNK)
    dst_fill = DST_PAD + (jnp.arange(pad, dtype=jnp.int32) % (N_PAD - N))
    dst2d = jnp.concatenate([dst, dst_fill]).reshape(E_PAD // CHUNK, CHUNK)
    ea_pad = jnp.concatenate([edge_attr, jnp.zeros((pad, ED), jnp.float32)], axis=0)

    W_x = msg_W[:D]
    W_e = msg_W[D:]

    eap = _eap(ea_pad, W_e, msg_b.reshape(1, D))
    out, Y = _lin0(x, lin0_W, lin0_b.reshape(1, D), W_x)

    bi2d = gru_bi.reshape(1, 3 * D)
    bh2d = gru_bh.reshape(1, 3 * D)
    edge_call = _edge_call()
    for _ in range(3):
        agg2, deg2 = edge_call(Y, eap, src2d, dst2d)
        deg_col = jnp.swapaxes(deg2, 0, 1)
        out, Y = _gru(agg2, deg_col, out, gru_Wi, gru_Wh, bi2d, bh2d, W_x)

    v = _s2s(out, batch.astype(jnp.int32).reshape(N, 1),
             batch.astype(jnp.int32).reshape(1, N),
             lstm_Wi, lstm_Wh, lstm_b.reshape(1, 4 * D),
             mlp_W1, mlp_b1.reshape(1, D), mlp_W2, mlp_b2.reshape(1, D),
             mlp_W3, mlp_b3.reshape(1, 1))
    return v


# speedup vs baseline: 1.1652x; 1.1652x over previous
import functools

import jax
import jax.numpy as jnp
from jax import lax
from jax.experimental import pallas as pl
from jax.experimental.pallas import tpu as pltpu
from jax.experimental.pallas import tpu_sc as plsc

N = 10000
E = 160000
NF = 128
ED = 16
D = 128
B = 64

NW = 32
TILES = 16
CHUNK = 64
CPT = 80
E_PAD = NW * CPT * CHUNK
N_PAD = 10240
ROWS_PT = N_PAD // TILES
DST_PAD = N

NEG = -3.0e38


def _bdot(a, b):
    return jnp.dot(a.astype(jnp.bfloat16), b.astype(jnp.bfloat16),
                   preferred_element_type=jnp.float32)

def _eap_body(ea_ref, we_ref, b_ref, o_ref):
    o_ref[...] = _bdot(ea_ref[...], we_ref[...]) + b_ref[...]


def _eap(ea_pad, W_e, b2d):
    blk = 4096
    return pl.pallas_call(
        _eap_body,
        grid=(E_PAD // blk,),
        in_specs=[
            pl.BlockSpec((blk, ED), lambda i: (i, 0)),
            pl.BlockSpec((ED, D), lambda i: (0, 0)),
            pl.BlockSpec((1, D), lambda i: (0, 0)),
        ],
        out_specs=pl.BlockSpec((blk, D), lambda i: (i, 0)),
        out_shape=jax.ShapeDtypeStruct((E_PAD, D), jnp.float32),
    )(ea_pad, W_e, b2d)


def _lin0_body(x_ref, w_ref, b_ref, wx_ref, out_ref, y_ref):
    h = jnp.maximum(_bdot(x_ref[...], w_ref[...]) + b_ref[...], 0.0)
    out_ref[...] = h
    y_ref[...] = _bdot(h, wx_ref[...])


def _lin0(x, lin0_W, b2d, W_x):
    blk = 2000
    return pl.pallas_call(
        _lin0_body,
        grid=(N // blk,),
        in_specs=[
            pl.BlockSpec((blk, NF), lambda i: (i, 0)),
            pl.BlockSpec((NF, D), lambda i: (0, 0)),
            pl.BlockSpec((1, D), lambda i: (0, 0)),
            pl.BlockSpec((D, D), lambda i: (0, 0)),
        ],
        out_specs=[
            pl.BlockSpec((blk, D), lambda i: (i, 0)),
            pl.BlockSpec((blk, D), lambda i: (i, 0)),
        ],
        out_shape=[
            jax.ShapeDtypeStruct((N, D), jnp.float32),
            jax.ShapeDtypeStruct((N, D), jnp.float32),
        ],
    )(x, lin0_W, b2d, W_x)


def _edge_body(y_hbm, eap_hbm, src_hbm, dst_hbm, agg_hbm,
               idx_s, idx_d, rows0, rows1, eapb,
               agg_sh, sg0, sg1, se):
    c = lax.axis_index("c")
    s = lax.axis_index("s")
    w = s * 2 + c
    t = s

    zero16 = jnp.zeros((16,), jnp.float32)

    @plsc.parallel_loop(0, CHUNK)
    def _zrow(r):
        for c8 in range(8):
            rows0[r, pl.ds(c8 * 16, 16)] = zero16

    for k in range(ROWS_PT // CHUNK):
        pltpu.sync_copy(rows0, agg_sh.at[pl.ds(t * ROWS_PT + k * CHUNK, CHUNK)])
    plsc.subcore_barrier()

    pltpu.sync_copy(src_hbm.at[pl.ds(w * CPT, CPT)], idx_s)
    pltpu.sync_copy(dst_hbm.at[pl.ds(w * CPT, CPT)], idx_d)

    rbufs = ((rows0, sg0), (rows1, sg1))

    def _start_g(j, b):
        rows, sg = rbufs[b]
        pltpu.async_copy(y_hbm.at[idx_s.at[j]], rows, sg)

    def _start_e(j):
        pltpu.async_copy(eap_hbm.at[pl.ds(w * (CPT * CHUNK) + j * CHUNK, CHUNK)],
                         eapb, se)

    _start_g(0, 0)
    _start_e(0)
    _start_g(1, 1)

    def _pair(jj, _):
        for b in range(2):
            j = jj * 2 + b
            rows, sg = rbufs[b]
            pltpu.make_async_copy(y_hbm.at[idx_s.at[j]], rows, sg).wait()
            pltpu.make_async_copy(
                eap_hbm.at[pl.ds(w * (CPT * CHUNK) + j * CHUNK, CHUNK)],
                eapb, se).wait()

            @plsc.parallel_loop(0, CHUNK, unroll=2)
            def _row(r):
                for c8 in range(8):
                    sl = pl.ds(c8 * 16, 16)
                    rows[r, sl] = jnp.maximum(rows[r, sl] + eapb[r, sl], 0.0)

            @pl.when(j + 1 < CPT)
            def _():
                _start_e(j + 1)

            pltpu.sync_copy(rows, agg_sh.at[idx_d.at[j]], add=True)

            @pl.when(j + 2 < CPT)
            def _():
                _start_g(j + 2, b)
        return 0

    lax.fori_loop(0, CPT // 2, _pair, 0)
    plsc.subcore_barrier()

    pltpu.sync_copy(agg_sh.at[pl.ds(t * ROWS_PT, ROWS_PT)],
                    agg_hbm.at[c, pl.ds(t * ROWS_PT, ROWS_PT)])


@functools.cache
def _edge_call():
    return functools.partial(
        pl.kernel,
        out_type=jax.ShapeDtypeStruct((2, N_PAD, D), jnp.float32),
        mesh=plsc.VectorSubcoreMesh(core_axis_name="c", subcore_axis_name="s",
                                    num_cores=2, num_subcores=TILES),
        scratch_types=[
        pltpu.VMEM((CPT, CHUNK), jnp.int32),
        pltpu.VMEM((CPT, CHUNK), jnp.int32),
        pltpu.VMEM((CHUNK, D), jnp.float32),
        pltpu.VMEM((CHUNK, D), jnp.float32),
        pltpu.VMEM((CHUNK, D), jnp.float32),
        pltpu.VMEM_SHARED((N_PAD, D), jnp.float32),
        pltpu.SemaphoreType.DMA,
        pltpu.SemaphoreType.DMA,
        pltpu.SemaphoreType.DMA,
        ],
    )(_edge_body)


def _deg_body(dst_hbm, deg_hbm, idx_d, ones, dzero, deg_sh):
    c = lax.axis_index("c")
    s = lax.axis_index("s")
    w = s * 2 + c
    t = s

    zero16 = jnp.zeros((16,), jnp.float32)
    for i in range(CHUNK // 16):
        ones[pl.ds(i * 16, 16)] = zero16 + 1.0

    @plsc.parallel_loop(0, ROWS_PT // 16)
    def _zd(i):
        dzero[pl.ds(i * 16, 16)] = zero16

    pltpu.sync_copy(dzero, deg_sh.at[pl.ds(t * ROWS_PT, ROWS_PT)])
    plsc.subcore_barrier()

    pltpu.sync_copy(dst_hbm.at[pl.ds(w * CPT, CPT)], idx_d)

    def _chunk(j, _):
        pltpu.sync_copy(ones, deg_sh.at[idx_d.at[j]], add=True)
        return 0

    lax.fori_loop(0, CPT, _chunk, 0)
    plsc.subcore_barrier()

    pltpu.sync_copy(deg_sh.at[pl.ds(t * ROWS_PT, ROWS_PT)],
                    deg_hbm.at[c, pl.ds(t * ROWS_PT, ROWS_PT)])


@functools.cache
def _deg_call():
    return functools.partial(
        pl.kernel,
        out_type=jax.ShapeDtypeStruct((2, N_PAD), jnp.float32),
        mesh=plsc.VectorSubcoreMesh(core_axis_name="c", subcore_axis_name="s",
                                    num_cores=2, num_subcores=TILES),
        scratch_types=[
        pltpu.VMEM((CPT, CHUNK), jnp.int32),
        pltpu.VMEM((CHUNK,), jnp.float32),
        pltpu.VMEM((ROWS_PT,), jnp.float32),
        pltpu.VMEM_SHARED((N_PAD,), jnp.float32),
        ],
    )(_deg_body)


def _gru_body(agg_ref, deg_ref, out_ref, wi_ref, wh_ref, bi_ref, bh_ref,
              wx_ref, new_ref, y_ref):
    d = deg_ref[...]
    inv = 1.0 / jnp.maximum(d[:, 0:1] + d[:, 1:2], 1.0)
    agg = (agg_ref[0] + agg_ref[1]) * inv
    out = out_ref[...]
    gi = _bdot(agg, wi_ref[...]) + bi_ref[...]
    gh = _bdot(out, wh_ref[...]) + bh_ref[...]
    r = jax.nn.sigmoid(gi[:, :D] + gh[:, :D])
    z = jax.nn.sigmoid(gi[:, D:2 * D] + gh[:, D:2 * D])
    n = jnp.tanh(gi[:, 2 * D:] + r * gh[:, 2 * D:])
    new = (1.0 - z) * n + z * out
    new_ref[...] = new
    y_ref[...] = _bdot(new, wx_ref[...])


def _gru(agg2, deg_col, out, gru_Wi, gru_Wh, bi2d, bh2d, W_x):
    blk = 2000
    return pl.pallas_call(
        _gru_body,
        grid=(N // blk,),
        in_specs=[
            pl.BlockSpec((2, blk, D), lambda i: (0, i, 0)),
            pl.BlockSpec((blk, 2), lambda i: (i, 0)),
            pl.BlockSpec((blk, D), lambda i: (i, 0)),
            pl.BlockSpec((D, 3 * D), lambda i: (0, 0)),
            pl.BlockSpec((D, 3 * D), lambda i: (0, 0)),
            pl.BlockSpec((1, 3 * D), lambda i: (0, 0)),
            pl.BlockSpec((1, 3 * D), lambda i: (0, 0)),
            pl.BlockSpec((D, D), lambda i: (0, 0)),
        ],
        out_specs=[
            pl.BlockSpec((blk, D), lambda i: (i, 0)),
            pl.BlockSpec((blk, D), lambda i: (i, 0)),
        ],
        out_shape=[
            jax.ShapeDtypeStruct((N, D), jnp.float32),
            jax.ShapeDtypeStruct((N, D), jnp.float32),
        ],
    )(agg2, deg_col, out, gru_Wi, gru_Wh, bi2d, bh2d, W_x)


def _s2s_body(out_ref, bcol_ref, brow_ref, lwi_ref, lwh_ref, lb_ref,
              w1_ref, b1_ref, w2_ref, b2_ref, w3_ref, b3_ref, v_ref):
    out = out_ref[...]
    bcol = bcol_ref[...]
    brow = brow_ref[...]
    onehot = (bcol == lax.broadcasted_iota(jnp.int32, (N, B), 1)).astype(jnp.float32)
    onehot_T = (brow == lax.broadcasted_iota(jnp.int32, (B, N), 0)).astype(jnp.float32)

    q_star = jnp.zeros((B, 2 * D), jnp.float32)
    h_l = jnp.zeros((B, D), jnp.float32)
    c_l = jnp.zeros((B, D), jnp.float32)
    dn_last = (((1,), (1,)), ((), ()))
    for _ in range(6):
        g = (_bdot(q_star, lwi_ref[...]) + _bdot(h_l, lwh_ref[...])
             + lb_ref[...])
        c_l = (jax.nn.sigmoid(g[:, D:2 * D]) * c_l
               + jax.nn.sigmoid(g[:, :D]) * jnp.tanh(g[:, 2 * D:3 * D]))
        h_l = jax.nn.sigmoid(g[:, 3 * D:]) * jnp.tanh(c_l)
        q = h_l
        e_mat = lax.dot_general(out, q, dn_last,
                                preferred_element_type=jnp.float32,
                                precision=lax.Precision.HIGHEST)
        e = jnp.sum(e_mat * onehot, axis=1, keepdims=True)
        seg_max = jnp.max(jnp.where(onehot > 0.0, e_mat, NEG), axis=0,
                          keepdims=True)
        e_max_b = lax.dot_general(onehot, seg_max, dn_last,
                                  preferred_element_type=jnp.float32,
                                  precision=lax.Precision.HIGHEST)
        a = jnp.exp(e - e_max_b)
        a_den = jnp.sum(onehot * a, axis=0, keepdims=True)
        a = a / lax.dot_general(onehot, a_den, dn_last,
                                preferred_element_type=jnp.float32,
                                precision=lax.Precision.HIGHEST)
        r_read = jnp.dot(onehot_T, a * out,
                         preferred_element_type=jnp.float32,
                         precision=lax.Precision.HIGHEST)
        q_star = jnp.concatenate([q, r_read], axis=1)

    v = jnp.maximum(_bdot(q_star, w1_ref[...]) + b1_ref[...], 0.0)
    v = jnp.maximum(_bdot(v, w2_ref[...]) + b2_ref[...], 0.0)
    v_ref[...] = _bdot(v, w3_ref[...]) + b3_ref[...]


def _s2s(out, bcol, brow, lstm_Wi, lstm_Wh, lb2d, w1, b1, w2, b2, w3, b3):
    return pl.pallas_call(
        _s2s_body,
        out_shape=jax.ShapeDtypeStruct((B, 1), jnp.float32),
    )(out, bcol, brow, lstm_Wi, lstm_Wh, lb2d, w1, b1, w2, b2, w3, b3)


def kernel(x, edge_index, edge_attr, batch, lin0_W, lin0_b, msg_W, msg_b,
           gru_Wi, gru_Wh, gru_bi, gru_bh, lstm_Wi, lstm_Wh, lstm_b,
           mlp_W1, mlp_b1, mlp_W2, mlp_b2, mlp_W3, mlp_b3):
    src = edge_index[0].astype(jnp.int32)
    dst = edge_index[1].astype(jnp.int32)
    pad = E_PAD - E
    src2d = jnp.concatenate([src, jnp.zeros((pad,), jnp.int32)]).reshape(E_PAD // CHUNK, CHUNK)
    dst_fill = DST_PAD + (jnp.arange(pad, dtype=jnp.int32) % (N_PAD - N))
    dst2d = jnp.concatenate([dst, dst_fill]).reshape(E_PAD // CHUNK, CHUNK)
    ea_pad = jnp.concatenate([edge_attr, jnp.zeros((pad, ED), jnp.float32)], axis=0)

    W_x = msg_W[:D]
    W_e = msg_W[D:]

    eap = _eap(ea_pad, W_e, msg_b.reshape(1, D))
    out, Y = _lin0(x, lin0_W, lin0_b.reshape(1, D), W_x)

    bi2d = gru_bi.reshape(1, 3 * D)
    bh2d = gru_bh.reshape(1, 3 * D)
    deg2 = _deg_call()(dst2d)
    deg_col = jnp.swapaxes(deg2, 0, 1)
    edge_call = _edge_call()
    for _ in range(3):
        agg2 = edge_call(Y, eap, src2d, dst2d)
        out, Y = _gru(agg2, deg_col, out, gru_Wi, gru_Wh, bi2d, bh2d, W_x)

    v = _s2s(out, batch.astype(jnp.int32).reshape(N, 1),
             batch.astype(jnp.int32).reshape(1, N),
             lstm_Wi, lstm_Wh, lstm_b.reshape(1, 4 * D),
             mlp_W1, mlp_b1.reshape(1, D), mlp_W2, mlp_b2.reshape(1, D),
             mlp_W3, mlp_b3.reshape(1, 1))
    return v


# speedup vs baseline: 2.1299x; 1.8279x over previous
import functools

import jax
import jax.numpy as jnp
from jax import lax
from jax.experimental import pallas as pl
from jax.experimental.pallas import tpu as pltpu
from jax.experimental.pallas import tpu_sc as plsc

N = 10000
E = 160000
NF = 128
ED = 16
D = 128
B = 64

NW = 32
TILES = 16
CHUNK = 64
CPT = 80
CPT_A = 48
CPT_B = CPT - CPT_A
CPT_MAX = max(CPT_A, CPT_B)
E_PAD = NW * CPT * CHUNK
N_PAD = 10240
ROWS_PT = N_PAD // TILES
DST_PAD = N

NEG = -3.0e38


def _bdot(a, b):
    return jnp.dot(a.astype(jnp.bfloat16), b.astype(jnp.bfloat16),
                   preferred_element_type=jnp.float32)

def _eap_body(ea_ref, we_ref, b_ref, o_ref):
    o_ref[...] = _bdot(ea_ref[...], we_ref[...]) + b_ref[...]


def _eap(ea_pad, W_e, b2d):
    blk = 4096
    return pl.pallas_call(
        _eap_body,
        grid=(E_PAD // blk,),
        in_specs=[
            pl.BlockSpec((blk, ED), lambda i: (i, 0)),
            pl.BlockSpec((ED, D), lambda i: (0, 0)),
            pl.BlockSpec((1, D), lambda i: (0, 0)),
        ],
        out_specs=pl.BlockSpec((blk, D), lambda i: (i, 0)),
        out_shape=jax.ShapeDtypeStruct((E_PAD, D), jnp.float32),
    )(ea_pad, W_e, b2d)


def _lin0_body(x_ref, w_ref, b_ref, wx_ref, out_ref, y_ref):
    h = jnp.maximum(_bdot(x_ref[...], w_ref[...]) + b_ref[...], 0.0)
    out_ref[...] = h
    y_ref[...] = _bdot(h, wx_ref[...])


def _lin0(x, lin0_W, b2d, W_x):
    blk = 2000
    return pl.pallas_call(
        _lin0_body,
        grid=(N // blk,),
        in_specs=[
            pl.BlockSpec((blk, NF), lambda i: (i, 0)),
            pl.BlockSpec((NF, D), lambda i: (0, 0)),
            pl.BlockSpec((1, D), lambda i: (0, 0)),
            pl.BlockSpec((D, D), lambda i: (0, 0)),
        ],
        out_specs=[
            pl.BlockSpec((blk, D), lambda i: (i, 0)),
            pl.BlockSpec((blk, D), lambda i: (i, 0)),
        ],
        out_shape=[
            jax.ShapeDtypeStruct((N, D), jnp.float32),
            jax.ShapeDtypeStruct((N, D), jnp.float32),
        ],
    )(x, lin0_W, b2d, W_x)


def _edge_body(y_hbm, eap_hbm, src_hbm, dst_hbm, agg_hbm,
               idx_s, idx_d, rows0, rows1, eapb,
               agg_sh, sg0, sg1, se):
    c = lax.axis_index("c")
    s = lax.axis_index("s")
    t = s
    my_cpt = jnp.where(c == 0, CPT_A, CPT_B)
    gbase = s * CPT + jnp.where(c == 0, 0, CPT_A)

    zero16 = jnp.zeros((16,), jnp.float32)

    @plsc.parallel_loop(0, CHUNK)
    def _zrow(r):
        for c8 in range(8):
            rows0[r, pl.ds(c8 * 16, 16)] = zero16

    for k in range(ROWS_PT // CHUNK):
        pltpu.sync_copy(rows0, agg_sh.at[pl.ds(t * ROWS_PT + k * CHUNK, CHUNK)])
    plsc.subcore_barrier()

    @pl.when(c == 0)
    def _():
        pltpu.sync_copy(src_hbm.at[pl.ds(gbase, CPT_A)], idx_s.at[pl.ds(0, CPT_A)])
        pltpu.sync_copy(dst_hbm.at[pl.ds(gbase, CPT_A)], idx_d.at[pl.ds(0, CPT_A)])

    @pl.when(c == 1)
    def _():
        pltpu.sync_copy(src_hbm.at[pl.ds(gbase, CPT_B)], idx_s.at[pl.ds(0, CPT_B)])
        pltpu.sync_copy(dst_hbm.at[pl.ds(gbase, CPT_B)], idx_d.at[pl.ds(0, CPT_B)])

    rbufs = ((rows0, sg0), (rows1, sg1))

    def _start_g(j, b):
        rows, sg = rbufs[b]
        pltpu.async_copy(y_hbm.at[idx_s.at[j]], rows, sg)

    def _start_e(j):
        pltpu.async_copy(eap_hbm.at[pl.ds((gbase + j) * CHUNK, CHUNK)], eapb, se)

    _start_g(0, 0)
    _start_e(0)

    @pl.when(1 < my_cpt)
    def _():
        _start_g(1, 1)

    def _pair(jj, _):
        for b in range(2):
            j = jj * 2 + b

            @pl.when(j < my_cpt)
            def _():
                rows, sg = rbufs[b]
                pltpu.make_async_copy(y_hbm.at[idx_s.at[j]], rows, sg).wait()
                pltpu.make_async_copy(
                    eap_hbm.at[pl.ds((gbase + j) * CHUNK, CHUNK)],
                    eapb, se).wait()

                @plsc.parallel_loop(0, CHUNK, unroll=2)
                def _row(r):
                    for c8 in range(8):
                        sl = pl.ds(c8 * 16, 16)
                        rows[r, sl] = jnp.maximum(rows[r, sl] + eapb[r, sl], 0.0)

                @pl.when(j + 1 < my_cpt)
                def _():
                    _start_e(j + 1)

                pltpu.sync_copy(rows, agg_sh.at[idx_d.at[j]], add=True)

                @pl.when(j + 2 < my_cpt)
                def _():
                    _start_g(j + 2, b)
        return 0

    lax.fori_loop(0, CPT_MAX // 2, _pair, 0)
    plsc.subcore_barrier()

    pltpu.sync_copy(agg_sh.at[pl.ds(t * ROWS_PT, ROWS_PT)],
                    agg_hbm.at[c, pl.ds(t * ROWS_PT, ROWS_PT)])


@functools.cache
def _edge_call():
    return functools.partial(
        pl.kernel,
        out_type=jax.ShapeDtypeStruct((2, N_PAD, D), jnp.float32),
        mesh=plsc.VectorSubcoreMesh(core_axis_name="c", subcore_axis_name="s",
                                    num_cores=2, num_subcores=TILES),
        scratch_types=[
        pltpu.VMEM((CPT_MAX, CHUNK), jnp.int32),
        pltpu.VMEM((CPT_MAX, CHUNK), jnp.int32),
        pltpu.VMEM((CHUNK, D), jnp.float32),
        pltpu.VMEM((CHUNK, D), jnp.float32),
        pltpu.VMEM((CHUNK, D), jnp.float32),
        pltpu.VMEM_SHARED((N_PAD, D), jnp.float32),
        pltpu.SemaphoreType.DMA,
        pltpu.SemaphoreType.DMA,
        pltpu.SemaphoreType.DMA,
        ],
    )(_edge_body)


def _deg_body(dst_hbm, deg_hbm, idx_d, ones, dzero, deg_sh):
    c = lax.axis_index("c")
    s = lax.axis_index("s")
    w = s * 2 + c
    t = s

    zero16 = jnp.zeros((16,), jnp.float32)
    for i in range(CHUNK // 16):
        ones[pl.ds(i * 16, 16)] = zero16 + 1.0

    @plsc.parallel_loop(0, ROWS_PT // 16)
    def _zd(i):
        dzero[pl.ds(i * 16, 16)] = zero16

    pltpu.sync_copy(dzero, deg_sh.at[pl.ds(t * ROWS_PT, ROWS_PT)])
    plsc.subcore_barrier()

    pltpu.sync_copy(dst_hbm.at[pl.ds(w * CPT, CPT)], idx_d)

    def _chunk(j, _):
        pltpu.sync_copy(ones, deg_sh.at[idx_d.at[j]], add=True)
        return 0

    lax.fori_loop(0, CPT, _chunk, 0)
    plsc.subcore_barrier()

    pltpu.sync_copy(deg_sh.at[pl.ds(t * ROWS_PT, ROWS_PT)],
                    deg_hbm.at[c, pl.ds(t * ROWS_PT, ROWS_PT)])


@functools.cache
def _deg_call():
    return functools.partial(
        pl.kernel,
        out_type=jax.ShapeDtypeStruct((2, N_PAD), jnp.float32),
        mesh=plsc.VectorSubcoreMesh(core_axis_name="c", subcore_axis_name="s",
                                    num_cores=2, num_subcores=TILES),
        scratch_types=[
        pltpu.VMEM((CPT, CHUNK), jnp.int32),
        pltpu.VMEM((CHUNK,), jnp.float32),
        pltpu.VMEM((ROWS_PT,), jnp.float32),
        pltpu.VMEM_SHARED((N_PAD,), jnp.float32),
        ],
    )(_deg_body)


def _gru_body(agg_ref, deg_ref, out_ref, wi_ref, wh_ref, bi_ref, bh_ref,
              wx_ref, new_ref, y_ref):
    d = deg_ref[...]
    inv = 1.0 / jnp.maximum(d[:, 0:1] + d[:, 1:2], 1.0)
    agg = (agg_ref[0] + agg_ref[1]) * inv
    out = out_ref[...]
    gi = _bdot(agg, wi_ref[...]) + bi_ref[...]
    gh = _bdot(out, wh_ref[...]) + bh_ref[...]
    r = jax.nn.sigmoid(gi[:, :D] + gh[:, :D])
    z = jax.nn.sigmoid(gi[:, D:2 * D] + gh[:, D:2 * D])
    n = jnp.tanh(gi[:, 2 * D:] + r * gh[:, 2 * D:])
    new = (1.0 - z) * n + z * out
    new_ref[...] = new
    y_ref[...] = _bdot(new, wx_ref[...])


def _gru(agg2, deg_col, out, gru_Wi, gru_Wh, bi2d, bh2d, W_x):
    blk = 2000
    return pl.pallas_call(
        _gru_body,
        grid=(N // blk,),
        in_specs=[
            pl.BlockSpec((2, blk, D), lambda i: (0, i, 0)),
            pl.BlockSpec((blk, 2), lambda i: (i, 0)),
            pl.BlockSpec((blk, D), lambda i: (i, 0)),
            pl.BlockSpec((D, 3 * D), lambda i: (0, 0)),
            pl.BlockSpec((D, 3 * D), lambda i: (0, 0)),
            pl.BlockSpec((1, 3 * D), lambda i: (0, 0)),
            pl.BlockSpec((1, 3 * D), lambda i: (0, 0)),
            pl.BlockSpec((D, D), lambda i: (0, 0)),
        ],
        out_specs=[
            pl.BlockSpec((blk, D), lambda i: (i, 0)),
            pl.BlockSpec((blk, D), lambda i: (i, 0)),
        ],
        out_shape=[
            jax.ShapeDtypeStruct((N, D), jnp.float32),
            jax.ShapeDtypeStruct((N, D), jnp.float32),
        ],
    )(agg2, deg_col, out, gru_Wi, gru_Wh, bi2d, bh2d, W_x)


def _s2s_body(out_ref, bcol_ref, brow_ref, lwi_ref, lwh_ref, lb_ref,
              w1_ref, b1_ref, w2_ref, b2_ref, w3_ref, b3_ref, v_ref):
    out = out_ref[...]
    bcol = bcol_ref[...]
    brow = brow_ref[...]
    onehot = (bcol == lax.broadcasted_iota(jnp.int32, (N, B), 1)).astype(jnp.float32)
    onehot_T = (brow == lax.broadcasted_iota(jnp.int32, (B, N), 0)).astype(jnp.float32)

    q_star = jnp.zeros((B, 2 * D), jnp.float32)
    h_l = jnp.zeros((B, D), jnp.float32)
    c_l = jnp.zeros((B, D), jnp.float32)
    dn_last = (((1,), (1,)), ((), ()))
    for _ in range(6):
        g = (_bdot(q_star, lwi_ref[...]) + _bdot(h_l, lwh_ref[...])
             + lb_ref[...])
        c_l = (jax.nn.sigmoid(g[:, D:2 * D]) * c_l
               + jax.nn.sigmoid(g[:, :D]) * jnp.tanh(g[:, 2 * D:3 * D]))
        h_l = jax.nn.sigmoid(g[:, 3 * D:]) * jnp.tanh(c_l)
        q = h_l
        e_mat = lax.dot_general(out, q, dn_last,
                                preferred_element_type=jnp.float32,
                                precision=lax.Precision.HIGHEST)
        e = jnp.sum(e_mat * onehot, axis=1, keepdims=True)
        seg_max = jnp.max(jnp.where(onehot > 0.0, e_mat, NEG), axis=0,
                          keepdims=True)
        e_max_b = lax.dot_general(onehot, seg_max, dn_last,
                                  preferred_element_type=jnp.float32,
                                  precision=lax.Precision.HIGHEST)
        a = jnp.exp(e - e_max_b)
        a_den = jnp.sum(onehot * a, axis=0, keepdims=True)
        a = a / lax.dot_general(onehot, a_den, dn_last,
                                preferred_element_type=jnp.float32,
                                precision=lax.Precision.HIGHEST)
        r_read = jnp.dot(onehot_T, a * out,
                         preferred_element_type=jnp.float32,
                         precision=lax.Precision.HIGHEST)
        q_star = jnp.concatenate([q, r_read], axis=1)

    v = jnp.maximum(_bdot(q_star, w1_ref[...]) + b1_ref[...], 0.0)
    v = jnp.maximum(_bdot(v, w2_ref[...]) + b2_ref[...], 0.0)
    v_ref[...] = _bdot(v, w3_ref[...]) + b3_ref[...]


def _s2s(out, bcol, brow, lstm_Wi, lstm_Wh, lb2d, w1, b1, w2, b2, w3, b3):
    return pl.pallas_call(
        _s2s_body,
        out_shape=jax.ShapeDtypeStruct((B, 1), jnp.float32),
    )(out, bcol, brow, lstm_Wi, lstm_Wh, lb2d, w1, b1, w2, b2, w3, b3)


def kernel(x, edge_index, edge_attr, batch, lin0_W, lin0_b, msg_W, msg_b,
           gru_Wi, gru_Wh, gru_bi, gru_bh, lstm_Wi, lstm_Wh, lstm_b,
           mlp_W1, mlp_b1, mlp_W2, mlp_b2, mlp_W3, mlp_b3):
    src = edge_index[0].astype(jnp.int32)
    dst = edge_index[1].astype(jnp.int32)
    pad = E_PAD - E
    src2d = jnp.concatenate([src, jnp.zeros((pad,), jnp.int32)]).reshape(E_PAD // CHUNK, CHUNK)
    dst_fill = DST_PAD + (jnp.arange(pad, dtype=jnp.int32) % (N_PAD - N))
    dst2d = jnp.concatenate([dst, dst_fill]).reshape(E_PAD // CHUNK, CHUNK)
    ea_pad = jnp.concatenate([edge_attr, jnp.zeros((pad, ED), jnp.float32)], axis=0)

    W_x = msg_W[:D]
    W_e = msg_W[D:]

    eap = _eap(ea_pad, W_e, msg_b.reshape(1, D))
    out, Y = _lin0(x, lin0_W, lin0_b.reshape(1, D), W_x)

    bi2d = gru_bi.reshape(1, 3 * D)
    bh2d = gru_bh.reshape(1, 3 * D)
    deg2 = _deg_call()(dst2d)
    deg_col = jnp.swapaxes(deg2, 0, 1)
    edge_call = _edge_call()
    for _ in range(3):
        agg2 = edge_call(Y, eap, src2d, dst2d)
        out, Y = _gru(agg2, deg_col, out, gru_Wi, gru_Wh, bi2d, bh2d, W_x)

    v = _s2s(out, batch.astype(jnp.int32).reshape(N, 1),
             batch.astype(jnp.int32).reshape(1, N),
             lstm_Wi, lstm_Wh, lstm_b.reshape(1, 4 * D),
             mlp_W1, mlp_b1.reshape(1, D), mlp_W2, mlp_b2.reshape(1, D),
             mlp_W3, mlp_b3.reshape(1, 1))
    return v
